# baseline (device time: 119920 ns/iter reference)
import jax
import jax.numpy as jnp
from jax import lax
from jax.experimental import pallas as pl
from jax.experimental.pallas import tpu as pltpu

N_DEV = 16


def kernel(table, idx):
    v_per, d = table.shape
    n = idx.shape[0]
    idx2 = idx.reshape(n, 1)

    def body(idx_ref, table_ref, out_ref, comm_ref, send_sems, recv_sems):
        my = lax.axis_index("i")
        left = (my - 1) % N_DEV
        right = (my + 1) % N_DEV

        barrier_sem = pltpu.get_barrier_semaphore()
        for nbr in (left, right):
            pl.semaphore_signal(
                barrier_sem, inc=1,
                device_id=(nbr,), device_id_type=pl.DeviceIdType.MESH,
            )
        pl.semaphore_wait(barrier_sem, 2)

        local = idx_ref[:, :] - my * v_per
        col = lax.broadcasted_iota(jnp.int32, (n, v_per), 1)
        oh = (col == local).astype(jnp.float32)
        partial = jnp.dot(oh, table_ref[:, :], preferred_element_type=jnp.float32)
        out_ref[:, :] = partial
        comm_ref[0] = partial

        for h in range(N_DEV - 1):
            rdma = pltpu.make_async_remote_copy(
                src_ref=comm_ref.at[h],
                dst_ref=comm_ref.at[h + 1],
                send_sem=send_sems.at[h],
                recv_sem=recv_sems.at[h],
                device_id=(right,),
                device_id_type=pl.DeviceIdType.MESH,
            )
            rdma.start()
            rdma.wait()
            out_ref[:, :] += comm_ref[h + 1]

    return pl.pallas_call(
        body,
        out_shape=jax.ShapeDtypeStruct((n, d), jnp.float32),
        in_specs=[
            pl.BlockSpec(memory_space=pltpu.VMEM),
            pl.BlockSpec(memory_space=pltpu.VMEM),
        ],
        out_specs=pl.BlockSpec(memory_space=pltpu.VMEM),
        scratch_shapes=[
            pltpu.VMEM((N_DEV, n, d), jnp.float32),
            pltpu.SemaphoreType.DMA((N_DEV - 1,)),
            pltpu.SemaphoreType.DMA((N_DEV - 1,)),
        ],
        compiler_params=pltpu.CompilerParams(collective_id=0),
    )(idx2, table)


# device time: 20672 ns/iter; 5.8011x vs baseline; 5.8011x over previous
import jax
import jax.numpy as jnp
from jax import lax
from jax.experimental import pallas as pl
from jax.experimental.pallas import tpu as pltpu

N_DEV = 16


def kernel(table, idx):
    v_per, d = table.shape
    n = idx.shape[0]
    m = n // N_DEV
    idx2 = idx.reshape(n, 1)

    def body(idx_ref, table_ref, out_ref, part_ref, rs_buf,
             rs_send_sems, rs_recv_sems, ag_send_sems, ag_recv_sems):
        my = lax.axis_index("i")

        barrier_sem = pltpu.get_barrier_semaphore()
        for r in range(1, N_DEV):
            peer = (my + r) % N_DEV
            pl.semaphore_signal(
                barrier_sem, inc=1,
                device_id=(peer,), device_id_type=pl.DeviceIdType.MESH,
            )
        pl.semaphore_wait(barrier_sem, N_DEV - 1)

        local = idx_ref[:, :] - my * v_per
        col = lax.broadcasted_iota(jnp.int32, (n, v_per), 1)
        oh = (col == local).astype(jnp.float32)
        part_ref[:, :] = jnp.dot(
            oh, table_ref[:, :], preferred_element_type=jnp.float32
        )

        rs_sends = []
        for r in range(1, N_DEV):
            peer = (my + r) % N_DEV
            k = N_DEV - r
            rdma = pltpu.make_async_remote_copy(
                src_ref=part_ref.at[pl.ds(peer * m, m), :],
                dst_ref=rs_buf.at[k],
                send_sem=rs_send_sems.at[r - 1],
                recv_sem=rs_recv_sems.at[k - 1],
                device_id=(peer,),
                device_id_type=pl.DeviceIdType.MESH,
            )
            rdma.start()
            rs_sends.append(rdma)

        rs_buf[0] = part_ref[pl.ds(my * m, m), :]

        for k in range(1, N_DEV):
            pltpu.make_async_remote_copy(
                src_ref=rs_buf.at[k],
                dst_ref=rs_buf.at[k],
                send_sem=rs_send_sems.at[k - 1],
                recv_sem=rs_recv_sems.at[k - 1],
                device_id=(my,),
                device_id_type=pl.DeviceIdType.MESH,
            ).wait_recv()
        red = jnp.sum(rs_buf[:, :, :], axis=0)
        out_ref[pl.ds(my * m, m), :] = red

        ag_sends = []
        for r in range(1, N_DEV):
            peer = (my + r) % N_DEV
            k = N_DEV - r
            rdma = pltpu.make_async_remote_copy(
                src_ref=out_ref.at[pl.ds(my * m, m), :],
                dst_ref=out_ref.at[pl.ds(my * m, m), :],
                send_sem=ag_send_sems.at[r - 1],
                recv_sem=ag_recv_sems.at[k - 1],
                device_id=(peer,),
                device_id_type=pl.DeviceIdType.MESH,
            )
            rdma.start()
            ag_sends.append(rdma)

        for k in range(1, N_DEV):
            src = (my + k) % N_DEV
            pltpu.make_async_remote_copy(
                src_ref=out_ref.at[pl.ds(src * m, m), :],
                dst_ref=out_ref.at[pl.ds(src * m, m), :],
                send_sem=ag_send_sems.at[k - 1],
                recv_sem=ag_recv_sems.at[k - 1],
                device_id=(my,),
                device_id_type=pl.DeviceIdType.MESH,
            ).wait_recv()

        for rdma in rs_sends:
            rdma.wait_send()
        for rdma in ag_sends:
            rdma.wait_send()

    return pl.pallas_call(
        body,
        out_shape=jax.ShapeDtypeStruct((n, d), jnp.float32),
        in_specs=[
            pl.BlockSpec(memory_space=pltpu.VMEM),
            pl.BlockSpec(memory_space=pltpu.VMEM),
        ],
        out_specs=pl.BlockSpec(memory_space=pltpu.VMEM),
        scratch_shapes=[
            pltpu.VMEM((n, d), jnp.float32),
            pltpu.VMEM((N_DEV, m, d), jnp.float32),
            pltpu.SemaphoreType.DMA((N_DEV - 1,)),
            pltpu.SemaphoreType.DMA((N_DEV - 1,)),
            pltpu.SemaphoreType.DMA((N_DEV - 1,)),
            pltpu.SemaphoreType.DMA((N_DEV - 1,)),
        ],
        compiler_params=pltpu.CompilerParams(collective_id=0),
    )(idx2, table)
